# baseline (speedup 1.0000x reference)
import functools

import jax
import jax.numpy as jnp
from jax import lax
from jax.experimental import pallas as pl
from jax.experimental.pallas import tpu as pltpu
from jax.experimental.pallas import tpu_sc as plsc

V = 100000
D = 64
B = 1024
C = 50
BH = B // 2

NC = 2
NS = 16
NW = NC * NS
DPW = D // NW
BBLK = 128
NBBLK = BH // BBLK
NSTEPS = DPW * NBBLK

_mesh = plsc.VectorSubcoreMesh(core_axis_name="c", subcore_axis_name="s")

_sc_scratch = [
    pltpu.VMEM((V,), jnp.float32),
    pltpu.VMEM((C, BBLK), jnp.int32),
    pltpu.VMEM((C, BBLK), jnp.int32),
    pltpu.VMEM((BH,), jnp.float32),
    pltpu.SemaphoreType.DMA,
    pltpu.SemaphoreType.DMA,
    pltpu.SemaphoreType.DMA,
]


def _make_gather_mean_t(b0):
    @functools.partial(
        pl.kernel,
        out_type=jax.ShapeDtypeStruct((D, BH), jnp.float32),
        mesh=_mesh,
        scratch_types=_sc_scratch,
        compiler_params=pltpu.CompilerParams(needs_layout_passes=False),
    )
    def _gather_mean_t(cwt_hbm, embt_hbm, out_hbm, row_v, cw_a, cw_b,
                       orow_v, sem_row, sem_a, sem_b):
        wid = lax.axis_index("s") * NC + lax.axis_index("c")
        inv = jnp.float32(1.0 / C)
        bufs = (cw_a, cw_b)
        sems = (sem_a, sem_b)

        def fire_cw(step):
            return pltpu.async_copy(
                cwt_hbm.at[:, pl.ds(b0 + (step % NBBLK) * BBLK, BBLK)],
                bufs[step % 2],
                sems[step % 2],
            )

        row_cp = pltpu.async_copy(embt_hbm.at[wid * DPW], row_v, sem_row)
        pend = {0: fire_cw(0), 1: fire_cw(1)}

        for p in range(DPW):
            d = wid * DPW + p
            row_cp.wait()
            for bblk in range(NBBLK):
                i = p * NBBLK + bblk
                pend.pop(i).wait()
                cw_v = bufs[i % 2]

                def c_body(c, accs):
                    new = []
                    for g in range(BBLK // 16):
                        idx16 = cw_v[c, pl.ds(g * 16, 16)]
                        vals = plsc.load_gather(row_v, [idx16])
                        new.append(accs[g] + vals)
                    return tuple(new)

                z = jnp.zeros((16,), jnp.float32)
                accs = lax.fori_loop(0, C, c_body, (z,) * (BBLK // 16))
                for g in range(BBLK // 16):
                    orow_v[pl.ds(bblk * BBLK + g * 16, 16)] = accs[g] * inv

                if i + 2 < NSTEPS:
                    pend[i + 2] = fire_cw(i + 2)
                if i == NBBLK - 1 and p + 1 < DPW:
                    row_cp = pltpu.async_copy(
                        embt_hbm.at[wid * DPW + p + 1], row_v, sem_row
                    )

            pltpu.sync_copy(orow_v, out_hbm.at[d])

    return _gather_mean_t


_gm_a = _make_gather_mean_t(0)
_gm_b = _make_gather_mean_t(BH)

VB = 4096


def _mm_body(meant_ref, wt_ref, b_ref, out_ref):
    acc = lax.dot_general(
        wt_ref[...],
        meant_ref[...],
        (((0,), (0,)), ((), ())),
        preferred_element_type=jnp.float32,
    )
    out_ref[...] = acc + b_ref[...][:, None]


def _mm_body_alias(meant_ref, wt_ref, b_ref, prev_ref, out_ref):
    del prev_ref
    _mm_body(meant_ref, wt_ref, b_ref, out_ref)


def _project_half(mean_t, Wt, bias, half, prev=None):
    nvb = pl.cdiv(V, VB)
    in_specs = [
        pl.BlockSpec((D, BH), lambda j: (0, 0)),
        pl.BlockSpec((D, VB), lambda j: (0, j)),
        pl.BlockSpec((VB,), lambda j: (j,)),
    ]
    args = (mean_t, Wt, bias)
    kwargs = {}
    body = _mm_body
    if prev is not None:
        in_specs.append(pl.BlockSpec(memory_space=pl.ANY))
        args += (prev,)
        kwargs = dict(input_output_aliases={3: 0})
        body = _mm_body_alias
    return pl.pallas_call(
        body,
        grid=(nvb,),
        in_specs=in_specs,
        out_specs=pl.BlockSpec((VB, BH), lambda j, h=half: (j, h)),
        out_shape=jax.ShapeDtypeStruct((V, B), jnp.float32),
        compiler_params=pltpu.CompilerParams(
            dimension_semantics=("arbitrary",),
        ),
        **kwargs,
    )(*args)


def kernel(context_words, emb_table, W, b):
    cw_t = context_words.T.astype(jnp.int32)
    embt = emb_table.T
    mean_a = _gm_a(cw_t, embt)
    mean_b = _gm_b(cw_t, embt)
    out1 = _project_half(mean_a, W.T, b, 0)
    out = _project_half(mean_b, W.T, b, 1, prev=out1)
    return out.T


# speedup vs baseline: 1.1334x; 1.1334x over previous
import functools

import jax
import jax.numpy as jnp
from jax import lax
from jax.experimental import pallas as pl
from jax.experimental.pallas import tpu as pltpu
from jax.experimental.pallas import tpu_sc as plsc

V = 100000
D = 64
B = 1024
C = 50

NC = 2
NS = 16
NW = NC * NS
DPW = D // NW
BBLK = 128
NBBLK = B // BBLK
CWCHUNK = BBLK * C

_mesh = plsc.VectorSubcoreMesh(core_axis_name="c", subcore_axis_name="s")


@functools.partial(
    pl.kernel,
    out_type=jax.ShapeDtypeStruct((D, B), jnp.float32),
    mesh=_mesh,
    scratch_types=[
        pltpu.VMEM((V,), jnp.float32),
        pltpu.VMEM((C, BBLK), jnp.int32),
        pltpu.VMEM((C, BBLK), jnp.int32),
        pltpu.VMEM((B,), jnp.float32),
        pltpu.SemaphoreType.DMA,
        pltpu.SemaphoreType.DMA,
        pltpu.SemaphoreType.DMA,
    ],
    compiler_params=pltpu.CompilerParams(needs_layout_passes=False),
)
def _gather_mean_t(cwt_hbm, embt_hbm, out_hbm, row_v, cw_a, cw_b, orow_v,
                   sem_row, sem_a, sem_b):
    wid = lax.axis_index("s") * NC + lax.axis_index("c")
    inv = jnp.float32(1.0 / C)
    bufs = (cw_a, cw_b)
    sems = (sem_a, sem_b)

    def fire_cw(step):
        return pltpu.async_copy(
            cwt_hbm.at[:, pl.ds((step % NBBLK) * BBLK, BBLK)],
            bufs[step % 2],
            sems[step % 2],
        )

    nsteps = DPW * NBBLK
    row_cp = pltpu.async_copy(embt_hbm.at[wid * DPW], row_v, sem_row)
    pend = {0: fire_cw(0), 1: fire_cw(1)}

    for p in range(DPW):
        d = wid * DPW + p
        row_cp.wait()
        for bblk in range(NBBLK):
            i = p * NBBLK + bblk
            pend.pop(i).wait()
            cw_v = bufs[i % 2]

            def c_body(c, accs):
                new = []
                for g in range(BBLK // 16):
                    idx16 = cw_v[c, pl.ds(g * 16, 16)]
                    vals = plsc.load_gather(row_v, [idx16])
                    new.append(accs[g] + vals)
                return tuple(new)

            z = jnp.zeros((16,), jnp.float32)
            accs = lax.fori_loop(0, C, c_body, (z,) * (BBLK // 16))
            for g in range(BBLK // 16):
                orow_v[pl.ds(bblk * BBLK + g * 16, 16)] = accs[g] * inv

            if i + 2 < nsteps:
                pend[i + 2] = fire_cw(i + 2)
            if i == NBBLK - 1 and p + 1 < DPW:
                row_cp = pltpu.async_copy(
                    embt_hbm.at[wid * DPW + p + 1], row_v, sem_row
                )

        pltpu.sync_copy(orow_v, out_hbm.at[d])


VB = 4096


def _mm_body(meant_ref, wt_ref, b_ref, out_ref):
    acc = lax.dot_general(
        wt_ref[...],
        meant_ref[...],
        (((0,), (0,)), ((), ())),
        preferred_element_type=jnp.float32,
    )
    out_ref[...] = acc + b_ref[...][:, None]


def _project_t(mean_t, Wt, b):
    nvb = pl.cdiv(V, VB)
    return pl.pallas_call(
        _mm_body,
        grid=(nvb,),
        in_specs=[
            pl.BlockSpec((D, B), lambda j: (0, 0)),
            pl.BlockSpec((D, VB), lambda j: (0, j)),
            pl.BlockSpec((VB,), lambda j: (j,)),
        ],
        out_specs=pl.BlockSpec((VB, B), lambda j: (j, 0)),
        out_shape=jax.ShapeDtypeStruct((V, B), jnp.float32),
        compiler_params=pltpu.CompilerParams(
            dimension_semantics=("arbitrary",),
        ),
    )(mean_t, Wt, b)


def kernel(context_words, emb_table, W, b):
    cw_t = context_words.T.astype(jnp.int32)
    mean_t = _gather_mean_t(cw_t, emb_table.T)
    out_t = _project_t(mean_t, W.T, b)
    return out_t.T


# speedup vs baseline: 1.1334x; 1.0001x over previous
import functools

import jax
import jax.numpy as jnp
from jax import lax
from jax.experimental import pallas as pl
from jax.experimental.pallas import tpu as pltpu
from jax.experimental.pallas import tpu_sc as plsc

V = 100000
D = 64
B = 1024
C = 50

NC = 2
NS = 16
NW = NC * NS
DPW = D // NW
BBLK = 128
NBBLK = B // BBLK
CWCHUNK = BBLK * C

_mesh = plsc.VectorSubcoreMesh(core_axis_name="c", subcore_axis_name="s")


@functools.partial(
    pl.kernel,
    out_type=jax.ShapeDtypeStruct((D, B), jnp.float32),
    mesh=_mesh,
    scratch_types=[
        pltpu.VMEM((V,), jnp.float32),
        pltpu.VMEM((C, BBLK), jnp.int32),
        pltpu.VMEM((C, BBLK), jnp.int32),
        pltpu.VMEM((B,), jnp.float32),
        pltpu.SemaphoreType.DMA,
        pltpu.SemaphoreType.DMA,
        pltpu.SemaphoreType.DMA,
    ],
    compiler_params=pltpu.CompilerParams(needs_layout_passes=False),
)
def _gather_mean_t(cwt_hbm, embt_hbm, out_hbm, row_v, cw_a, cw_b, orow_v,
                   sem_row, sem_a, sem_b):
    wid = lax.axis_index("s") * NC + lax.axis_index("c")
    inv = jnp.float32(1.0 / C)
    bufs = (cw_a, cw_b)
    sems = (sem_a, sem_b)

    def fire_cw(step):
        return pltpu.async_copy(
            cwt_hbm.at[:, pl.ds((step % NBBLK) * BBLK, BBLK)],
            bufs[step % 2],
            sems[step % 2],
        )

    nsteps = DPW * NBBLK
    row_cp = pltpu.async_copy(embt_hbm.at[wid * DPW], row_v, sem_row)
    pend = {0: fire_cw(0), 1: fire_cw(1)}

    for p in range(DPW):
        d = wid * DPW + p
        row_cp.wait()
        for bblk in range(NBBLK):
            i = p * NBBLK + bblk
            pend.pop(i).wait()
            cw_v = bufs[i % 2]

            def c_body(c, accs):
                new = []
                for g in range(BBLK // 16):
                    idx16 = cw_v[c, pl.ds(g * 16, 16)]
                    vals = plsc.load_gather(row_v, [idx16])
                    new.append(accs[g] + vals)
                return tuple(new)

            z = jnp.zeros((16,), jnp.float32)
            accs = lax.fori_loop(0, C, c_body, (z,) * (BBLK // 16))
            for g in range(BBLK // 16):
                orow_v[pl.ds(bblk * BBLK + g * 16, 16)] = accs[g] * inv

            if i + 2 < nsteps:
                pend[i + 2] = fire_cw(i + 2)
            if i == NBBLK - 1 and p + 1 < DPW:
                row_cp = pltpu.async_copy(
                    embt_hbm.at[wid * DPW + p + 1], row_v, sem_row
                )

        pltpu.sync_copy(orow_v, out_hbm.at[d])


VB = 5120


def _mm_body(meant_ref, wt_ref, b_ref, out_ref):
    acc = lax.dot_general(
        wt_ref[...],
        meant_ref[...],
        (((0,), (0,)), ((), ())),
        preferred_element_type=jnp.float32,
    )
    out_ref[...] = acc + b_ref[...][:, None]


def _project_t(mean_t, Wt, b):
    nvb = pl.cdiv(V, VB)
    return pl.pallas_call(
        _mm_body,
        grid=(nvb,),
        in_specs=[
            pl.BlockSpec((D, B), lambda j: (0, 0)),
            pl.BlockSpec((D, VB), lambda j: (0, j)),
            pl.BlockSpec((VB,), lambda j: (j,)),
        ],
        out_specs=pl.BlockSpec((VB, B), lambda j: (j, 0)),
        out_shape=jax.ShapeDtypeStruct((V, B), jnp.float32),
        compiler_params=pltpu.CompilerParams(
            dimension_semantics=("arbitrary",),
        ),
    )(mean_t, Wt, b)


def kernel(context_words, emb_table, W, b):
    cw_t = context_words.T.astype(jnp.int32)
    mean_t = _gather_mean_t(cw_t, emb_table.T)
    out_t = _project_t(mean_t, W.T, b)
    return out_t.T


# speedup vs baseline: 1.1383x; 1.0042x over previous
import functools

import jax
import jax.numpy as jnp
from jax import lax
from jax.experimental import pallas as pl
from jax.experimental.pallas import tpu as pltpu
from jax.experimental.pallas import tpu_sc as plsc

V = 100000
D = 64
B = 1024
C = 50

NC = 2
NS = 16
NW = NC * NS
DPW = D // NW
BBLK = 128
NBBLK = B // BBLK
CWCHUNK = BBLK * C

_mesh = plsc.VectorSubcoreMesh(core_axis_name="c", subcore_axis_name="s")


@functools.partial(
    pl.kernel,
    out_type=jax.ShapeDtypeStruct((D, B), jnp.float32),
    mesh=_mesh,
    scratch_types=[
        pltpu.VMEM((V,), jnp.float32),
        pltpu.VMEM((C, BBLK), jnp.int32),
        pltpu.VMEM((C, BBLK), jnp.int32),
        pltpu.VMEM((B,), jnp.float32),
        pltpu.SemaphoreType.DMA,
        pltpu.SemaphoreType.DMA,
        pltpu.SemaphoreType.DMA,
    ],
    compiler_params=pltpu.CompilerParams(needs_layout_passes=False),
)
def _gather_mean_t(cwt_hbm, embt_hbm, out_hbm, row_v, cw_a, cw_b, orow_v,
                   sem_row, sem_a, sem_b):
    wid = lax.axis_index("s") * NC + lax.axis_index("c")
    inv = jnp.float32(1.0 / C)
    bufs = (cw_a, cw_b)
    sems = (sem_a, sem_b)

    def fire_cw(step):
        return pltpu.async_copy(
            cwt_hbm.at[:, pl.ds((step % NBBLK) * BBLK, BBLK)],
            bufs[step % 2],
            sems[step % 2],
        )

    nsteps = DPW * NBBLK
    row_cp = pltpu.async_copy(embt_hbm.at[wid * DPW], row_v, sem_row)
    pend = {0: fire_cw(0), 1: fire_cw(1)}

    for p in range(DPW):
        d = wid * DPW + p
        row_cp.wait()
        for bblk in range(NBBLK):
            i = p * NBBLK + bblk
            pend.pop(i).wait()
            cw_v = bufs[i % 2]

            def c_body(c, accs):
                new = []
                for g in range(BBLK // 16):
                    idx16 = cw_v[c, pl.ds(g * 16, 16)]
                    vals = plsc.load_gather(row_v, [idx16])
                    new.append(accs[g] + vals)
                return tuple(new)

            z = jnp.zeros((16,), jnp.float32)
            accs = lax.fori_loop(0, C, c_body, (z,) * (BBLK // 16))
            for g in range(BBLK // 16):
                orow_v[pl.ds(bblk * BBLK + g * 16, 16)] = accs[g] * inv

            if i + 2 < nsteps:
                pend[i + 2] = fire_cw(i + 2)
            if i == NBBLK - 1 and p + 1 < DPW:
                row_cp = pltpu.async_copy(
                    embt_hbm.at[wid * DPW + p + 1], row_v, sem_row
                )

        pltpu.sync_copy(orow_v, out_hbm.at[d])


VB = 4096


def _mm_body(meant_ref, wt_ref, b_ref, out_ref):
    acc = lax.dot_general(
        wt_ref[...],
        meant_ref[...],
        (((0,), (0,)), ((), ())),
        preferred_element_type=jnp.float32,
    )
    out_ref[...] = acc + b_ref[...][:, None]


def _project_t(mean_t, Wt, b):
    nvb = pl.cdiv(V, VB)
    return pl.pallas_call(
        _mm_body,
        grid=(nvb,),
        in_specs=[
            pl.BlockSpec((D, B), lambda j: (0, 0)),
            pl.BlockSpec((D, VB), lambda j: (0, j)),
            pl.BlockSpec((VB,), lambda j: (j,)),
        ],
        out_specs=pl.BlockSpec((VB, B), lambda j: (j, 0)),
        out_shape=jax.ShapeDtypeStruct((V, B), jnp.float32),
        compiler_params=pltpu.CompilerParams(
            dimension_semantics=("arbitrary",),
        ),
    )(mean_t, Wt, b)


def kernel(context_words, emb_table, W, b):
    cw_t = context_words.T.astype(jnp.int32)
    mean_t = _gather_mean_t(cw_t, emb_table.T)
    out_t = _project_t(mean_t, W.T, b)
    return out_t.T
